# baseline (speedup 1.0000x reference)
import functools

import jax
import jax.numpy as jnp
from jax import lax
from jax.experimental import pallas as pl
from jax.experimental.pallas import tpu as pltpu
from jax.experimental.pallas import tpu_sc as plsc

N = 10000
E = 320000
H = 128
LANES = 16
NC = 2
NS = 16
NW = NC * NS

EMB_CHUNK = 80
EMB_NCHUNK = N // EMB_CHUNK
EDGE_CHUNK = 128
EDGE_NCHUNK = E // EDGE_CHUNK
CPB = 2
LCH = 160
HCH = 40
NB = LCH // CPB
PCH = NS * LCH
EPAD = PCH * EDGE_CHUNK
ROWS_PER_TILE = 624
ROWS_REMAIN = N - NS * ROWS_PER_TILE

_mesh = plsc.VectorSubcoreMesh(core_axis_name="c", subcore_axis_name="s")


_GDN = lax.GatherDimensionNumbers(
    offset_dims=(), collapsed_slice_dims=(0,), start_index_map=(0,))


def _lane_bcast(v16, lane):
    idx = jnp.full((LANES, 1), lane, jnp.int32)
    return lax.gather(v16, idx, _GDN, (1,),
                      mode=lax.GatherScatterMode.PROMISE_IN_BOUNDS)


@functools.partial(
    pl.kernel,
    mesh=_mesh,
    out_type=jax.ShapeDtypeStruct((N, H), jnp.float32),
    scratch_types=[
        pltpu.VMEM((EMB_CHUNK,), jnp.int32),
        pltpu.VMEM((EMB_CHUNK,), jnp.float32),
        pltpu.VMEM((EMB_CHUNK, H), jnp.float32),
        pltpu.VMEM((H,), jnp.float32),
        pltpu.SemaphoreType.DMA,
    ],
)
def _emb_call(feat_idx, psw, emb, bias, h0, idx_v, w_v, rows_v, bias_v, sem):
    wid = lax.axis_index("s") * NC + lax.axis_index("c")
    pltpu.sync_copy(bias, bias_v)
    nper = (EMB_NCHUNK + NW - 1) // NW
    for t in range(nper):
        j = wid + NW * t

        @pl.when(j < EMB_NCHUNK)
        def _():
            base = pl.multiple_of(j * EMB_CHUNK, EMB_CHUNK)
            pltpu.sync_copy(feat_idx.at[pl.ds(base, EMB_CHUNK)], idx_v)
            pltpu.sync_copy(psw.at[pl.ds(base, EMB_CHUNK)], w_v)
            pltpu.async_copy(emb.at[idx_v], rows_v, sem).wait()

            def body(g, carry):
                v16 = w_v[pl.ds(pl.multiple_of(g * LANES, LANES), LANES)]
                for lane in range(LANES):
                    w = _lane_bcast(v16, lane)
                    r = g * LANES + lane
                    for c in range(H // LANES):
                        sl = pl.ds(c * LANES, LANES)
                        rows_v[r, sl] = jnp.maximum(
                            rows_v[r, sl] * w + bias_v[sl], 0.0)
                return carry
            lax.fori_loop(0, EMB_CHUNK // LANES, body, 0)
            pltpu.sync_copy(rows_v, h0.at[pl.ds(base, EMB_CHUNK)])


@functools.partial(
    pl.kernel,
    mesh=_mesh,
    out_type=jax.ShapeDtypeStruct((2, N, H), jnp.float32),
    scratch_types=[
        pltpu.VMEM((HCH * EDGE_CHUNK,), jnp.int32),
        pltpu.VMEM((HCH, EDGE_CHUNK), jnp.int32),
        pltpu.VMEM((HCH * EDGE_CHUNK,), jnp.float32),
        pltpu.VMEM((EDGE_CHUNK, H), jnp.float32),
        pltpu.VMEM((EDGE_CHUNK, H), jnp.float32),
        pltpu.VMEM_SHARED((N, H), jnp.float32),
        pltpu.SemaphoreType.DMA,
        pltpu.SemaphoreType.DMA,
        pltpu.SemaphoreType.DMA,
        pltpu.SemaphoreType.DMA,
    ],
)
def _edge_call(h, src1, dstp, wflat, out2,
               src_all, dst_all, w_all, rows_a, rows_b, acc,
               sg_a, sg_b, ss_a, ss_b):
    cid = lax.axis_index("c")
    sid = lax.axis_index("s")
    rows = (rows_a, rows_b)
    sg = (sg_a, sg_b)
    ss = (ss_a, ss_b)

    def zbody(r, carry):
        for c in range(H // LANES):
            rows_a[r, pl.ds(c * LANES, LANES)] = jnp.zeros((LANES,), jnp.float32)
        return carry
    lax.fori_loop(0, EDGE_CHUNK, zbody, 0)
    row0 = sid * ROWS_PER_TILE
    for k in range(4):
        pltpu.sync_copy(rows_a, acc.at[pl.ds(row0 + k * EDGE_CHUNK, EDGE_CHUNK)])
    pltpu.sync_copy(rows_a.at[pl.ds(0, 112)], acc.at[pl.ds(row0 + 512, 112)])

    @pl.when(sid == NS - 1)
    def _():
        pltpu.sync_copy(rows_a.at[pl.ds(0, ROWS_REMAIN)],
                        acc.at[pl.ds(NS * ROWS_PER_TILE, ROWS_REMAIN)])
    plsc.subcore_barrier()

    def _wait_scatter(b):
        pltpu.make_async_copy(rows[b], acc.at[dst_all.at[0]], ss[b]).wait()

    def _compute(b, kp, guard):
        def go():
            pltpu.make_async_copy(h.at[src_all.at[pl.ds(0, EDGE_CHUNK)]],
                                  rows[b], sg[b]).wait()

            def sbody(g, carry):
                v16 = w_all[pl.ds(pl.multiple_of(kp * EDGE_CHUNK + g * LANES,
                                                 LANES), LANES)]
                for lane in range(LANES):
                    wv = _lane_bcast(v16, lane)
                    r = g * LANES + lane
                    for c in range(H // LANES):
                        sl = pl.ds(c * LANES, LANES)
                        rows[b][r, sl] = rows[b][r, sl] * wv
                return carry
            lax.fori_loop(0, EDGE_CHUNK // LANES, sbody, 0)
            pltpu.async_copy(rows[b], acc.at[dst_all.at[kp]], ss[b], add=True)
        if guard is None:
            go()
        else:
            pl.when(guard)(go)

    def hbody(hh, carry):
        g0 = pl.multiple_of(sid * LCH + hh * HCH, CPB)
        e0 = pl.multiple_of((sid * LCH + hh * HCH) * EDGE_CHUNK, EDGE_CHUNK)
        pltpu.sync_copy(src1.at[pl.ds(e0, HCH * EDGE_CHUNK)], src_all)
        pltpu.sync_copy(dstp.at[pl.ds(g0, HCH)], dst_all)
        pltpu.sync_copy(wflat.at[pl.ds(cid * EPAD + e0, HCH * EDGE_CHUNK)], w_all)

        def obody(o, carry2):
            for b in range(2):
                k = 2 * o + b

                @pl.when(k >= 2)
                def _(b=b):
                    _wait_scatter(b)
                base = pl.multiple_of(k * EDGE_CHUNK, EDGE_CHUNK)
                pltpu.async_copy(h.at[src_all.at[pl.ds(base, EDGE_CHUNK)]],
                                 rows[b], sg[b])
                _compute(1 - b, k - 1, k >= 1)
            return carry2
        lax.fori_loop(0, HCH // 2, obody, 0)
        _compute((HCH - 1) % 2, HCH - 1, None)
        _wait_scatter((HCH - 2) % 2)
        _wait_scatter((HCH - 1) % 2)
        return carry
    lax.fori_loop(0, LCH // HCH, hbody, 0)
    plsc.subcore_barrier()

    pltpu.sync_copy(acc.at[pl.ds(row0, ROWS_PER_TILE)],
                    out2.at[cid, pl.ds(row0, ROWS_PER_TILE)])

    @pl.when(sid == NS - 1)
    def _():
        pltpu.sync_copy(acc.at[pl.ds(NS * ROWS_PER_TILE, ROWS_REMAIN)],
                        out2.at[cid, pl.ds(NS * ROWS_PER_TILE, ROWS_REMAIN)])


BLK = 1000


def _layer_body(ppi_ref, res_ref, w_ref, b_ref, o_ref):
    z = lax.dot_general(ppi_ref[...], w_ref[...], (((1,), (1,)), ((), ())),
                        preferred_element_type=jnp.float32)
    o_ref[...] = jnp.maximum(z + b_ref[...], 0.0) + res_ref[...]


def _layer_update(ppi, res, W, b2d):
    return pl.pallas_call(
        _layer_body,
        grid=(N // BLK,),
        in_specs=[
            pl.BlockSpec((BLK, H), lambda i: (i, 0)),
            pl.BlockSpec((BLK, H), lambda i: (i, 0)),
            pl.BlockSpec((H, H), lambda i: (0, 0)),
            pl.BlockSpec((1, H), lambda i: (0, 0)),
        ],
        out_specs=pl.BlockSpec((BLK, H), lambda i: (i, 0)),
        out_shape=jax.ShapeDtypeStruct((N, H), jnp.float32),
    )(ppi, res, W, b2d)


def _final_body(ppi_ref, res_ref, w_ref, b_ref, wo_ref, bo_ref, o_ref):
    z = lax.dot_general(ppi_ref[...], w_ref[...], (((1,), (1,)), ((), ())),
                        preferred_element_type=jnp.float32)
    hcur = jnp.maximum(z + b_ref[...], 0.0) + res_ref[...]
    o_ref[...] = lax.dot_general(hcur, wo_ref[...], (((1,), (1,)), ((), ())),
                                 preferred_element_type=jnp.float32) + bo_ref[...]


def _final_update(ppi, res, W, b2d, wo_p, bo_p):
    return pl.pallas_call(
        _final_body,
        grid=(N // BLK,),
        in_specs=[
            pl.BlockSpec((BLK, H), lambda i: (i, 0)),
            pl.BlockSpec((BLK, H), lambda i: (i, 0)),
            pl.BlockSpec((H, H), lambda i: (0, 0)),
            pl.BlockSpec((1, H), lambda i: (0, 0)),
            pl.BlockSpec((H, H), lambda i: (0, 0)),
            pl.BlockSpec((1, H), lambda i: (0, 0)),
        ],
        out_specs=pl.BlockSpec((BLK, H), lambda i: (i, 0)),
        out_shape=jax.ShapeDtypeStruct((N, H), jnp.float32),
    )(ppi, res, W, b2d, wo_p, bo_p)


def kernel(feat_idx, offsets, per_sample_weights, edge_index, w_ppi, w_self,
           emb_table, input_bias, W1, b1, W2, b2, Wout, bout):
    del offsets
    eidx = edge_index.astype(jnp.int32)
    pad = EPAD - E
    src1 = jnp.pad(eidx[0], (0, pad))
    dstp = jnp.pad(eidx[1], (0, pad)).reshape(PCH, EDGE_CHUNK)
    wflat = jnp.concatenate([jnp.pad(w_ppi, (0, pad)),
                             jnp.pad(w_self, (0, pad))])
    h0 = _emb_call(feat_idx.astype(jnp.int32), per_sample_weights,
                   emb_table, input_bias)
    pair1 = _edge_call(h0, src1, dstp, wflat)
    h1 = _layer_update(pair1[0], pair1[1], W1, b1.reshape(1, H))
    pair2 = _edge_call(h1, src1, dstp, wflat)
    ppi2, res2 = pair2[0], pair2[1]
    C = Wout.shape[0]
    wo_p = jnp.zeros((H, H), jnp.float32).at[:C].set(Wout)
    bo_p = jnp.zeros((1, H), jnp.float32).at[0, :C].set(bout)
    out = _final_update(ppi2, res2, W2, b2.reshape(1, H), wo_p, bo_p)
    return out[:, :C]


# speedup vs baseline: 2.7785x; 2.7785x over previous
import functools

import jax
import jax.numpy as jnp
from jax import lax
from jax.experimental import pallas as pl
from jax.experimental.pallas import tpu as pltpu
from jax.experimental.pallas import tpu_sc as plsc

N = 10000
E = 320000
H = 128
LANES = 16
NC = 2
NS = 16
NW = NC * NS

EMB_CHUNK = 80
EMB_NCHUNK = N // EMB_CHUNK
EDGE_CHUNK = 128
EDGE_NCHUNK = E // EDGE_CHUNK
CPB = 2
LCH = 160
HCH = 40
NB = LCH // CPB
PCH = NS * LCH
EPAD = PCH * EDGE_CHUNK
ROWS_PER_TILE = 624
ROWS_REMAIN = N - NS * ROWS_PER_TILE

_mesh = plsc.VectorSubcoreMesh(core_axis_name="c", subcore_axis_name="s")


_GDN = lax.GatherDimensionNumbers(
    offset_dims=(), collapsed_slice_dims=(0,), start_index_map=(0,))


def _lane_bcast(v16, lane):
    idx = jnp.full((LANES, 1), lane, jnp.int32)
    return lax.gather(v16, idx, _GDN, (1,),
                      mode=lax.GatherScatterMode.PROMISE_IN_BOUNDS)


@functools.partial(
    pl.kernel,
    mesh=_mesh,
    out_type=jax.ShapeDtypeStruct((N, H), jnp.float32),
    scratch_types=[
        pltpu.VMEM((EMB_CHUNK,), jnp.int32),
        pltpu.VMEM((EMB_CHUNK,), jnp.float32),
        pltpu.VMEM((EMB_CHUNK, H), jnp.float32),
        pltpu.VMEM((H,), jnp.float32),
        pltpu.SemaphoreType.DMA,
    ],
)
def _emb_call(feat_idx, psw, emb, bias, h0, idx_v, w_v, rows_v, bias_v, sem):
    wid = lax.axis_index("s") * NC + lax.axis_index("c")
    pltpu.sync_copy(bias, bias_v)
    nper = (EMB_NCHUNK + NW - 1) // NW
    for t in range(nper):
        j = wid + NW * t

        @pl.when(j < EMB_NCHUNK)
        def _():
            base = pl.multiple_of(j * EMB_CHUNK, EMB_CHUNK)
            pltpu.sync_copy(feat_idx.at[pl.ds(base, EMB_CHUNK)], idx_v)
            pltpu.sync_copy(psw.at[pl.ds(base, EMB_CHUNK)], w_v)
            pltpu.async_copy(emb.at[idx_v], rows_v, sem).wait()

            def body(g, carry):
                v16 = w_v[pl.ds(pl.multiple_of(g * LANES, LANES), LANES)]
                for lane in range(LANES):
                    w = _lane_bcast(v16, lane)
                    r = g * LANES + lane
                    for c in range(H // LANES):
                        sl = pl.ds(c * LANES, LANES)
                        rows_v[r, sl] = jnp.maximum(
                            rows_v[r, sl] * w + bias_v[sl], 0.0)
                return carry
            lax.fori_loop(0, EMB_CHUNK // LANES, body, 0)
            pltpu.sync_copy(rows_v, h0.at[pl.ds(base, EMB_CHUNK)])


@functools.partial(
    pl.kernel,
    mesh=_mesh,
    out_type=jax.ShapeDtypeStruct((2, N, H), jnp.float32),
    scratch_types=[
        pltpu.VMEM((HCH * EDGE_CHUNK,), jnp.int32),
        pltpu.VMEM((HCH, EDGE_CHUNK), jnp.int32),
        pltpu.VMEM((HCH * EDGE_CHUNK,), jnp.float32),
        pltpu.VMEM((EDGE_CHUNK, H), jnp.float32),
        pltpu.VMEM((EDGE_CHUNK, H), jnp.float32),
        pltpu.VMEM_SHARED((N, H), jnp.float32),
        pltpu.SemaphoreType.DMA,
        pltpu.SemaphoreType.DMA,
        pltpu.SemaphoreType.DMA,
        pltpu.SemaphoreType.DMA,
    ],
)
def _edge_call(h, src1, dstp, wflat, out2,
               src_all, dst_all, w_all, rows_a, rows_b, acc,
               sg_a, sg_b, ss_a, ss_b):
    cid = lax.axis_index("c")
    sid = lax.axis_index("s")
    rows = (rows_a, rows_b)
    sg = (sg_a, sg_b)
    ss = (ss_a, ss_b)

    def zbody(r, carry):
        for c in range(H // LANES):
            rows_a[r, pl.ds(c * LANES, LANES)] = jnp.zeros((LANES,), jnp.float32)
        return carry
    lax.fori_loop(0, EDGE_CHUNK, zbody, 0)
    row0 = sid * ROWS_PER_TILE
    for k in range(4):
        pltpu.sync_copy(rows_a, acc.at[pl.ds(row0 + k * EDGE_CHUNK, EDGE_CHUNK)])
    pltpu.sync_copy(rows_a.at[pl.ds(0, 112)], acc.at[pl.ds(row0 + 512, 112)])

    @pl.when(sid == NS - 1)
    def _():
        pltpu.sync_copy(rows_a.at[pl.ds(0, ROWS_REMAIN)],
                        acc.at[pl.ds(NS * ROWS_PER_TILE, ROWS_REMAIN)])
    plsc.subcore_barrier()

    def _wait_scatter(b):
        pltpu.make_async_copy(rows[b], acc.at[dst_all.at[0]], ss[b]).wait()

    def _compute(b, kp, guard):
        def go():
            pltpu.make_async_copy(h.at[src_all.at[pl.ds(0, EDGE_CHUNK)]],
                                  rows[b], sg[b]).wait()

            def sbody(g, carry):
                v16 = w_all[pl.ds(pl.multiple_of(kp * EDGE_CHUNK + g * LANES,
                                                 LANES), LANES)]
                for lane in range(LANES):
                    wv = _lane_bcast(v16, lane)
                    r = g * LANES + lane
                    for c in range(H // LANES):
                        sl = pl.ds(c * LANES, LANES)
                        rows[b][r, sl] = rows[b][r, sl] * wv
                return carry
            lax.fori_loop(0, EDGE_CHUNK // LANES, sbody, 0)
            pltpu.async_copy(rows[b], acc.at[dst_all.at[kp]], ss[b], add=True)
        if guard is None:
            go()
        else:
            pl.when(guard)(go)

    def hbody(hh, carry):
        g0 = pl.multiple_of(sid * LCH + hh * HCH, CPB)
        e0 = pl.multiple_of((sid * LCH + hh * HCH) * EDGE_CHUNK, EDGE_CHUNK)
        pltpu.sync_copy(src1.at[pl.ds(e0, HCH * EDGE_CHUNK)], src_all)
        pltpu.sync_copy(dstp.at[pl.ds(g0, HCH)], dst_all)
        pltpu.sync_copy(wflat.at[pl.ds(cid * EPAD + e0, HCH * EDGE_CHUNK)], w_all)

        def obody(o, carry2):
            for b in range(2):
                k = 2 * o + b

                @pl.when(k >= 2)
                def _(b=b):
                    _wait_scatter(b)
                base = pl.multiple_of(k * EDGE_CHUNK, EDGE_CHUNK)
                pltpu.async_copy(h.at[src_all.at[pl.ds(base, EDGE_CHUNK)]],
                                 rows[b], sg[b])
                _compute(1 - b, k - 1, k >= 1)
            return carry2
        lax.fori_loop(0, HCH // 2, obody, 0)
        _compute((HCH - 1) % 2, HCH - 1, None)
        _wait_scatter((HCH - 2) % 2)
        _wait_scatter((HCH - 1) % 2)
        return carry
    lax.fori_loop(0, LCH // HCH, hbody, 0)
    plsc.subcore_barrier()

    pltpu.sync_copy(acc.at[pl.ds(row0, ROWS_PER_TILE)],
                    out2.at[cid, pl.ds(row0, ROWS_PER_TILE)])

    @pl.when(sid == NS - 1)
    def _():
        pltpu.sync_copy(acc.at[pl.ds(NS * ROWS_PER_TILE, ROWS_REMAIN)],
                        out2.at[cid, pl.ds(NS * ROWS_PER_TILE, ROWS_REMAIN)])


BLK = 1000


def _layer_body(ppi_ref, res_ref, w_ref, b_ref, o_ref):
    z = lax.dot_general(ppi_ref[...], w_ref[...], (((1,), (1,)), ((), ())),
                        preferred_element_type=jnp.float32)
    o_ref[...] = jnp.maximum(z + b_ref[...], 0.0) + res_ref[...]


def _layer_update(ppi, res, W, b2d):
    return pl.pallas_call(
        _layer_body,
        grid=(N // BLK,),
        in_specs=[
            pl.BlockSpec((BLK, H), lambda i: (i, 0)),
            pl.BlockSpec((BLK, H), lambda i: (i, 0)),
            pl.BlockSpec((H, H), lambda i: (0, 0)),
            pl.BlockSpec((1, H), lambda i: (0, 0)),
        ],
        out_specs=pl.BlockSpec((BLK, H), lambda i: (i, 0)),
        out_shape=jax.ShapeDtypeStruct((N, H), jnp.float32),
    )(ppi, res, W, b2d)


def _final_body(ppi_ref, res_ref, w_ref, b_ref, wo_ref, bo_ref, o_ref):
    z = lax.dot_general(ppi_ref[...], w_ref[...], (((1,), (1,)), ((), ())),
                        preferred_element_type=jnp.float32)
    hcur = jnp.maximum(z + b_ref[...], 0.0) + res_ref[...]
    o_ref[...] = lax.dot_general(hcur, wo_ref[...], (((1,), (1,)), ((), ())),
                                 preferred_element_type=jnp.float32) + bo_ref[...]


def _final_update(ppi, res, W, b2d, wo_p, bo_p):
    return pl.pallas_call(
        _final_body,
        grid=(N // BLK,),
        in_specs=[
            pl.BlockSpec((BLK, H), lambda i: (i, 0)),
            pl.BlockSpec((BLK, H), lambda i: (i, 0)),
            pl.BlockSpec((H, H), lambda i: (0, 0)),
            pl.BlockSpec((1, H), lambda i: (0, 0)),
            pl.BlockSpec((H, H), lambda i: (0, 0)),
            pl.BlockSpec((1, H), lambda i: (0, 0)),
        ],
        out_specs=pl.BlockSpec((BLK, H), lambda i: (i, 0)),
        out_shape=jax.ShapeDtypeStruct((N, H), jnp.float32),
    )(ppi, res, W, b2d, wo_p, bo_p)


def kernel(feat_idx, offsets, per_sample_weights, edge_index, w_ppi, w_self,
           emb_table, input_bias, W1, b1, W2, b2, Wout, bout):
    del offsets
    eidx = edge_index.astype(jnp.int32)
    pad = EPAD - E
    spread = (jnp.arange(pad, dtype=jnp.int32) * 37) % N
    src1 = jnp.concatenate([eidx[0], spread])
    dstp = jnp.concatenate([eidx[1], spread]).reshape(PCH, EDGE_CHUNK)
    wflat = jnp.concatenate([jnp.pad(w_ppi, (0, pad)),
                             jnp.pad(w_self, (0, pad))])
    h0 = _emb_call(feat_idx.astype(jnp.int32), per_sample_weights,
                   emb_table, input_bias)
    pair1 = _edge_call(h0, src1, dstp, wflat)
    h1 = _layer_update(pair1[0], pair1[1], W1, b1.reshape(1, H))
    pair2 = _edge_call(h1, src1, dstp, wflat)
    ppi2, res2 = pair2[0], pair2[1]
    C = Wout.shape[0]
    wo_p = jnp.zeros((H, H), jnp.float32).at[:C].set(Wout)
    bo_p = jnp.zeros((1, H), jnp.float32).at[0, :C].set(bout)
    out = _final_update(ppi2, res2, W2, b2.reshape(1, H), wo_p, bo_p)
    return out[:, :C]
